# baseline (speedup 1.0000x reference)
import functools

import jax
import jax.numpy as jnp
from jax import lax
from jax.experimental import pallas as pl
from jax.experimental.pallas import tpu as pltpu
from jax.experimental.pallas import tpu_sc as plsc

N = 10000
E = 160000
H = 256
OUT = 128
G = 64
BN_EPS = 1e-5

NC = 2
NS = 16
HALF = H // NC
CH = 128
CPB = 8
NBK = 10
EPAD = NS * NBK * CPB * CH
NDUM = 8
RPT = 624
TAIL = N - NS * RPT

BLK = 1000
NB = N // BLK


def _sc_agg_body(x2_hbm, src_hbm, dst_hbm, h2_hbm,
                 srcb0, srcb1, dstb0, dstb1, rows0, rows1,
                 acc, semg0, semg1, semi0, semi1, sems0, sems1):
    c = lax.axis_index("c")
    s = lax.axis_index("s")
    pltpu.sync_copy(x2_hbm.at[c, pl.ds(s * RPT, RPT)],
                    acc.at[pl.ds(s * RPT, RPT)])

    @pl.when(s == NS - 1)
    def _():
        pltpu.sync_copy(x2_hbm.at[c, pl.ds(NS * RPT, TAIL)],
                        acc.at[pl.ds(NS * RPT, TAIL)])

    xc = x2_hbm.at[c]
    srcb = (srcb0, srcb1)
    dstb = (dstb0, dstb1)
    rows = (rows0, rows1)
    semg = (semg0, semg1)
    semi = (semi0, semi1)
    sems = (sems0, sems1)

    def _idx_load(b, p):
        pltpu.async_copy(src_hbm.at[s, b], srcb[p], semi[p])
        pltpu.async_copy(dst_hbm.at[s, b], dstb[p], semi[p])

    def _idx_wait(b, p):
        pltpu.make_async_copy(src_hbm.at[s, b], srcb[p], semi[p]).wait()
        pltpu.make_async_copy(dst_hbm.at[s, b], dstb[p], semi[p]).wait()

    _idx_load(0, 0)
    _idx_load(1, 1)
    _idx_wait(0, 0)
    plsc.subcore_barrier()

    def _gather(p, k):
        pltpu.async_copy(xc.at[srcb[p].at[k]], rows[k % 2], semg[k % 2])

    def _gwait(p, k):
        pltpu.make_async_copy(xc.at[srcb[p].at[k]], rows[k % 2],
                              semg[k % 2]).wait()

    def _scat(p, k):
        pltpu.async_copy(rows[k % 2], acc.at[dstb[p].at[k]], sems[k % 2],
                         add=True)

    def _swait(p, k):
        pltpu.make_async_copy(rows[k % 2], acc.at[dstb[p].at[k]],
                              sems[k % 2]).wait()

    _gather(0, 0)

    def _pair(g, carry):
        for p in (0, 1):
            b = 2 * g + p
            q = 1 - p
            for k in range(CPB):
                w = k % 2
                if p == 0 and k == 0:
                    @pl.when(g > 0)
                    def _():
                        _swait(q, CPB - 1)
                        _idx_load(b + 1, q)
                elif p == 1 and k == 0:
                    _swait(q, CPB - 1)

                    @pl.when(g < NBK // 2 - 1)
                    def _():
                        _idx_load(b + 1, q)
                else:
                    _swait(p, k - 1)
                if k < CPB - 1:
                    _gather(p, k + 1)
                elif p == 0:
                    _idx_wait(b + 1, q)
                    _gather(q, 0)
                else:
                    @pl.when(g < NBK // 2 - 1)
                    def _():
                        _idx_wait(b + 1, q)
                        _gather(q, 0)
                _gwait(p, k)
                _scat(p, k)
        return carry

    lax.fori_loop(0, NBK // 2, _pair, 0)
    _swait(1, CPB - 1)

    plsc.subcore_barrier()
    pltpu.sync_copy(acc.at[pl.ds(s * RPT, RPT)],
                    h2_hbm.at[c, pl.ds(s * RPT, RPT)])

    @pl.when(s == NS - 1)
    def _():
        pltpu.sync_copy(acc.at[pl.ds(NS * RPT, TAIL)],
                        h2_hbm.at[c, pl.ds(NS * RPT, TAIL)])


@functools.cache
def _make_sc_agg():
    mesh = plsc.VectorSubcoreMesh(core_axis_name="c", subcore_axis_name="s",
                                  num_cores=NC, num_subcores=NS)
    return pl.kernel(
        _sc_agg_body,
        out_type=jax.ShapeDtypeStruct((NC, N, HALF), jnp.float32),
        mesh=mesh,
        scratch_types=[
            pltpu.VMEM((CPB, CH), jnp.int32),
            pltpu.VMEM((CPB, CH), jnp.int32),
            pltpu.VMEM((CPB, CH), jnp.int32),
            pltpu.VMEM((CPB, CH), jnp.int32),
            pltpu.VMEM((CH, HALF), jnp.float32),
            pltpu.VMEM((CH, HALF), jnp.float32),
            pltpu.VMEM_SHARED((N + NDUM, HALF), jnp.float32),
            pltpu.SemaphoreType.DMA,
            pltpu.SemaphoreType.DMA,
            pltpu.SemaphoreType.DMA,
            pltpu.SemaphoreType.DMA,
            pltpu.SemaphoreType.DMA,
            pltpu.SemaphoreType.DMA,
        ],
    )


def _k0_body(x_ref, o_ref):
    o_ref[0] = x_ref[:, :HALF]
    o_ref[1] = x_ref[:, HALF:]


def _split(x):
    return pl.pallas_call(
        _k0_body,
        grid=(NB,),
        in_specs=[pl.BlockSpec((BLK, H), lambda i: (i, 0))],
        out_specs=pl.BlockSpec((NC, BLK, HALF), lambda i: (0, i, 0)),
        out_shape=jax.ShapeDtypeStruct((NC, N, HALF), jnp.float32),
    )(x)


def _k1_body(h2_ref, w1_ref, b1_ref, h1_ref, st_ref):
    i = pl.program_id(0)
    h = jnp.concatenate([h2_ref[0], h2_ref[1]], axis=1)
    h1 = jnp.dot(h, w1_ref[...], preferred_element_type=jnp.float32)
    h1 = h1 + b1_ref[...]
    h1_ref[...] = h1
    ps = jnp.sum(h1, axis=0, keepdims=True)
    pq = jnp.sum(h1 * h1, axis=0, keepdims=True)
    prev_s = jnp.where(i == 0, 0.0, st_ref[0:1, :])
    prev_q = jnp.where(i == 0, 0.0, st_ref[1:2, :])
    st_ref[0:1, :] = prev_s + ps
    st_ref[1:2, :] = prev_q + pq


def _mlp_pass1(h2, w1, b1):
    return pl.pallas_call(
        _k1_body,
        grid=(NB,),
        in_specs=[
            pl.BlockSpec((NC, BLK, HALF), lambda i: (0, i, 0)),
            pl.BlockSpec((H, H), lambda i: (0, 0)),
            pl.BlockSpec((1, H), lambda i: (0, 0)),
        ],
        out_specs=[
            pl.BlockSpec((BLK, H), lambda i: (i, 0)),
            pl.BlockSpec((8, H), lambda i: (0, 0)),
        ],
        out_shape=[
            jax.ShapeDtypeStruct((N, H), jnp.float32),
            jax.ShapeDtypeStruct((8, H), jnp.float32),
        ],
        compiler_params=pltpu.CompilerParams(
            dimension_semantics=("arbitrary",)),
    )(h2, w1, b1)


def _k2_body(h1_ref, st_ref, g_ref, bt_ref, w2_ref, b2_ref, o_ref):
    mean = st_ref[0:1, :] * (1.0 / N)
    var = st_ref[1:2, :] * (1.0 / N) - mean * mean
    inv = lax.rsqrt(var + BN_EPS)
    scale = inv * g_ref[...]
    shift = bt_ref[...] - mean * scale
    hn = jnp.maximum(h1_ref[...] * scale + shift, 0.0)
    y = jnp.dot(hn, w2_ref[...], preferred_element_type=jnp.float32)
    y = jnp.maximum(y + b2_ref[...], 0.0)
    o_ref[0] = y[:, :HALF]
    o_ref[1] = y[:, HALF:]


def _mlp_pass2(h1, stats, gamma, beta, w2, b2):
    return pl.pallas_call(
        _k2_body,
        grid=(NB,),
        in_specs=[
            pl.BlockSpec((BLK, H), lambda i: (i, 0)),
            pl.BlockSpec((8, H), lambda i: (0, 0)),
            pl.BlockSpec((1, H), lambda i: (0, 0)),
            pl.BlockSpec((1, H), lambda i: (0, 0)),
            pl.BlockSpec((H, H), lambda i: (0, 0)),
            pl.BlockSpec((1, H), lambda i: (0, 0)),
        ],
        out_specs=pl.BlockSpec((NC, BLK, HALF), lambda i: (0, i, 0)),
        out_shape=jax.ShapeDtypeStruct((NC, N, HALF), jnp.float32),
        compiler_params=pltpu.CompilerParams(
            dimension_semantics=("arbitrary",)),
    )(h1, stats, gamma, beta, w2, b2)


def _kp_body(x2_ref, b_ref, wm1_ref, bm1_ref, wm2_ref, bm2_ref,
             out_ref, pooled_ref):
    i = pl.program_id(0)
    xb = jnp.concatenate([x2_ref[0], x2_ref[1]], axis=1)
    b = b_ref[0]
    gid = lax.broadcasted_iota(jnp.int32, (G, BLK), 0)
    onehot = jnp.where(b == gid, 1.0, 0.0)
    part = jnp.dot(onehot, xb, preferred_element_type=jnp.float32)
    acc = jnp.where(i == 0, 0.0, pooled_ref[...]) + part
    pooled_ref[...] = acc

    @pl.when(i == NB - 1)
    def _():
        hd = jnp.dot(acc, wm1_ref[...], preferred_element_type=jnp.float32)
        hd = jnp.maximum(hd + bm1_ref[...], 0.0)
        y = jnp.dot(hd, wm2_ref[...], preferred_element_type=jnp.float32)
        out_ref[...] = y + bm2_ref[...]


def _pool(x2, batch3, wm1, bm1, wm2, bm2):
    return pl.pallas_call(
        _kp_body,
        grid=(NB,),
        in_specs=[
            pl.BlockSpec((NC, BLK, HALF), lambda i: (0, i, 0)),
            pl.BlockSpec((1, 1, BLK), lambda i: (i, 0, 0)),
            pl.BlockSpec((H, H), lambda i: (0, 0)),
            pl.BlockSpec((1, H), lambda i: (0, 0)),
            pl.BlockSpec((H, OUT), lambda i: (0, 0)),
            pl.BlockSpec((1, OUT), lambda i: (0, 0)),
        ],
        out_specs=[
            pl.BlockSpec((G, OUT), lambda i: (0, 0)),
            pl.BlockSpec((G, H), lambda i: (0, 0)),
        ],
        out_shape=[
            jax.ShapeDtypeStruct((G, OUT), jnp.float32),
            jax.ShapeDtypeStruct((G, H), jnp.float32),
        ],
        compiler_params=pltpu.CompilerParams(
            dimension_semantics=("arbitrary",)),
    )(x2, batch3, wm1, bm1, wm2, bm2)


def _agg_call(x2, src3, dst3):
    return _make_sc_agg()(x2, src3, dst3)


def kernel(x, edge_index, edge_attr, batch, conv_params, mlp_params):
    npad = EPAD - E
    pad_ids = jnp.arange(npad, dtype=jnp.int32)
    src_p = jnp.concatenate([edge_index[0], pad_ids % N])
    dst_p = jnp.concatenate([edge_index[1], N + pad_ids % NDUM])
    src3 = src_p.reshape(NS, NBK, CPB, CH)
    dst3 = dst_p.reshape(NS, NBK, CPB, CH)
    batch3 = batch.reshape(NB, 1, BLK)

    x2 = _split(x)
    for (w1, b1, gamma, beta, w2, b2) in conv_params:
        h2 = _agg_call(x2, src3, dst3)
        h1, stats = _mlp_pass1(h2, w1, b1.reshape(1, H))
        x2 = _mlp_pass2(h1, stats, gamma.reshape(1, H), beta.reshape(1, H),
                        w2, b2.reshape(1, H))

    wm1, bm1, wm2, bm2 = mlp_params
    out, pooled = _pool(x2, batch3, wm1, bm1.reshape(1, H),
                        wm2, bm2.reshape(1, OUT))
    return out, pooled


# speedup vs baseline: 1.0818x; 1.0818x over previous
import functools

import jax
import jax.numpy as jnp
from jax import lax
from jax.experimental import pallas as pl
from jax.experimental.pallas import tpu as pltpu
from jax.experimental.pallas import tpu_sc as plsc

N = 10000
E = 160000
H = 256
OUT = 128
G = 64
BN_EPS = 1e-5

NC = 2
NS = 16
HALF = H // NC
CH = 128
CPB = 8
NBK = 10
EPAD = NS * NBK * CPB * CH
NDUM = 8
RPT = 624
TAIL = N - NS * RPT

BLK = 1000
NB = N // BLK


def _sc_agg_body(x2_hbm, src_hbm, dst_hbm, h2_hbm,
                 srcb0, srcb1, dstb0, dstb1, rows0, rows1,
                 acc, semg0, semg1, semi0, semi1, sems0, sems1):
    c = lax.axis_index("c")
    s = lax.axis_index("s")
    pltpu.sync_copy(x2_hbm.at[c, pl.ds(s * RPT, RPT)],
                    acc.at[pl.ds(s * RPT, RPT)])

    @pl.when(s == NS - 1)
    def _():
        pltpu.sync_copy(x2_hbm.at[c, pl.ds(NS * RPT, TAIL)],
                        acc.at[pl.ds(NS * RPT, TAIL)])

    xc = x2_hbm.at[c]
    srcb = (srcb0, srcb1)
    dstb = (dstb0, dstb1)
    rows = (rows0, rows1)
    semg = (semg0, semg1)
    semi = (semi0, semi1)
    sems = (sems0, sems1)

    def _idx_load(b, p):
        pltpu.async_copy(src_hbm.at[s, b], srcb[p], semi[p])
        pltpu.async_copy(dst_hbm.at[s, b], dstb[p], semi[p])

    def _idx_wait(b, p):
        pltpu.make_async_copy(src_hbm.at[s, b], srcb[p], semi[p]).wait()
        pltpu.make_async_copy(dst_hbm.at[s, b], dstb[p], semi[p]).wait()

    _idx_load(0, 0)
    _idx_load(1, 1)
    _idx_wait(0, 0)
    plsc.subcore_barrier()

    def _gather(p, k):
        pltpu.async_copy(xc.at[srcb[p].at[k]], rows[k % 2], semg[k % 2])

    def _gwait(p, k):
        pltpu.make_async_copy(xc.at[srcb[p].at[k]], rows[k % 2],
                              semg[k % 2]).wait()

    def _scat(p, k):
        pltpu.async_copy(rows[k % 2], acc.at[dstb[p].at[k]], sems[k % 2],
                         add=True)

    def _swait(p, k):
        pltpu.make_async_copy(rows[k % 2], acc.at[dstb[p].at[k]],
                              sems[k % 2]).wait()

    _gather(0, 0)

    def _pair(g, carry):
        for p in (0, 1):
            b = 2 * g + p
            q = 1 - p
            for k in range(CPB):
                w = k % 2
                if p == 0 and k == 0:
                    @pl.when(g > 0)
                    def _():
                        _swait(q, CPB - 1)
                        _idx_load(b + 1, q)
                elif p == 1 and k == 0:
                    _swait(q, CPB - 1)

                    @pl.when(g < NBK // 2 - 1)
                    def _():
                        _idx_load(b + 1, q)
                else:
                    _swait(p, k - 1)
                if k < CPB - 1:
                    _gather(p, k + 1)
                elif p == 0:
                    _idx_wait(b + 1, q)
                    _gather(q, 0)
                else:
                    @pl.when(g < NBK // 2 - 1)
                    def _():
                        _idx_wait(b + 1, q)
                        _gather(q, 0)
                _gwait(p, k)
                _scat(p, k)
        return carry

    lax.fori_loop(0, NBK // 2, _pair, 0)
    _swait(1, CPB - 1)

    plsc.subcore_barrier()
    pltpu.sync_copy(acc.at[pl.ds(s * RPT, RPT)],
                    h2_hbm.at[c, pl.ds(s * RPT, RPT)])

    @pl.when(s == NS - 1)
    def _():
        pltpu.sync_copy(acc.at[pl.ds(NS * RPT, TAIL)],
                        h2_hbm.at[c, pl.ds(NS * RPT, TAIL)])


@functools.cache
def _make_sc_agg():
    mesh = plsc.VectorSubcoreMesh(core_axis_name="c", subcore_axis_name="s",
                                  num_cores=NC, num_subcores=NS)
    return pl.kernel(
        _sc_agg_body,
        out_type=jax.ShapeDtypeStruct((NC, N, HALF), jnp.float32),
        mesh=mesh,
        scratch_types=[
            pltpu.VMEM((CPB, CH), jnp.int32),
            pltpu.VMEM((CPB, CH), jnp.int32),
            pltpu.VMEM((CPB, CH), jnp.int32),
            pltpu.VMEM((CPB, CH), jnp.int32),
            pltpu.VMEM((CH, HALF), jnp.float32),
            pltpu.VMEM((CH, HALF), jnp.float32),
            pltpu.VMEM_SHARED((N + NDUM, HALF), jnp.float32),
            pltpu.SemaphoreType.DMA,
            pltpu.SemaphoreType.DMA,
            pltpu.SemaphoreType.DMA,
            pltpu.SemaphoreType.DMA,
            pltpu.SemaphoreType.DMA,
            pltpu.SemaphoreType.DMA,
        ],
    )


def _k0_body(x_ref, o_ref):
    o_ref[0] = x_ref[:, :HALF]
    o_ref[1] = x_ref[:, HALF:]


def _split(x):
    return pl.pallas_call(
        _k0_body,
        grid=(NB,),
        in_specs=[pl.BlockSpec((BLK, H), lambda i: (i, 0))],
        out_specs=pl.BlockSpec((NC, BLK, HALF), lambda i: (0, i, 0)),
        out_shape=jax.ShapeDtypeStruct((NC, N, HALF), jnp.float32),
    )(x)


def _layer_phase0(h2_ref, w1_ref, b1_ref, h1_scr, st_scr, i):
    h = jnp.concatenate([h2_ref[0], h2_ref[1]], axis=1)
    h1 = jnp.dot(h, w1_ref[...], preferred_element_type=jnp.float32)
    h1 = h1 + b1_ref[...]
    h1_scr[pl.ds(i * BLK, BLK), :] = h1
    ps = jnp.sum(h1, axis=0, keepdims=True)
    pq = jnp.sum(h1 * h1, axis=0, keepdims=True)
    prev_s = jnp.where(i == 0, 0.0, st_scr[0:1, :])
    prev_q = jnp.where(i == 0, 0.0, st_scr[1:2, :])
    st_scr[0:1, :] = prev_s + ps
    st_scr[1:2, :] = prev_q + pq


def _layer_phase1_y(st_scr, g_ref, bt_ref, w2_ref, b2_ref, h1_scr, i):
    mean = st_scr[0:1, :] * (1.0 / N)
    var = st_scr[1:2, :] * (1.0 / N) - mean * mean
    inv = lax.rsqrt(var + BN_EPS)
    scale = inv * g_ref[...]
    shift = bt_ref[...] - mean * scale
    h1 = h1_scr[pl.ds(i * BLK, BLK), :]
    hn = jnp.maximum(h1 * scale + shift, 0.0)
    y = jnp.dot(hn, w2_ref[...], preferred_element_type=jnp.float32)
    return jnp.maximum(y + b2_ref[...], 0.0)


def _k12_body(h2_ref, w1_ref, b1_ref, g_ref, bt_ref, w2_ref, b2_ref,
              o_ref, h1_scr, st_scr):
    ph = pl.program_id(0)
    i = pl.program_id(1)

    @pl.when(ph == 0)
    def _():
        _layer_phase0(h2_ref, w1_ref, b1_ref, h1_scr, st_scr, i)

    @pl.when(ph == 1)
    def _():
        y = _layer_phase1_y(st_scr, g_ref, bt_ref, w2_ref, b2_ref, h1_scr, i)
        o_ref[0] = y[:, :HALF]
        o_ref[1] = y[:, HALF:]


def _const_spec(shape):
    return pl.BlockSpec(shape, lambda ph, i: (0,) * len(shape))


def _layer_tc(h2, w1, b1, gamma, beta, w2, b2):
    return pl.pallas_call(
        _k12_body,
        grid=(2, NB),
        in_specs=[
            pl.BlockSpec((NC, BLK, HALF),
                         lambda ph, i: (0, jnp.where(ph == 0, i, 0), 0)),
            _const_spec((H, H)),
            _const_spec((1, H)),
            _const_spec((1, H)),
            _const_spec((1, H)),
            _const_spec((H, H)),
            _const_spec((1, H)),
        ],
        out_specs=pl.BlockSpec((NC, BLK, HALF),
                               lambda ph, i: (0, jnp.where(ph == 1, i, 0), 0)),
        out_shape=jax.ShapeDtypeStruct((NC, N, HALF), jnp.float32),
        scratch_shapes=[
            pltpu.VMEM((N, H), jnp.float32),
            pltpu.VMEM((8, H), jnp.float32),
        ],
        compiler_params=pltpu.CompilerParams(
            dimension_semantics=("arbitrary", "arbitrary")),
    )(h2, w1, b1, gamma, beta, w2, b2)


def _k12p_body(h2_ref, b_ref, w1_ref, b1_ref, g_ref, bt_ref, w2_ref, b2_ref,
               wm1_ref, bm1_ref, wm2_ref, bm2_ref,
               out_ref, pooled_ref, h1_scr, st_scr):
    ph = pl.program_id(0)
    i = pl.program_id(1)

    @pl.when(ph == 0)
    def _():
        _layer_phase0(h2_ref, w1_ref, b1_ref, h1_scr, st_scr, i)

    @pl.when(ph == 1)
    def _():
        y = _layer_phase1_y(st_scr, g_ref, bt_ref, w2_ref, b2_ref, h1_scr, i)
        b = b_ref[0]
        gid = lax.broadcasted_iota(jnp.int32, (G, BLK), 0)
        onehot = jnp.where(b == gid, 1.0, 0.0)
        part = jnp.dot(onehot, y, preferred_element_type=jnp.float32)
        acc = jnp.where(i == 0, 0.0, pooled_ref[...]) + part
        pooled_ref[...] = acc

        @pl.when(i == NB - 1)
        def _():
            hd = jnp.dot(acc, wm1_ref[...],
                         preferred_element_type=jnp.float32)
            hd = jnp.maximum(hd + bm1_ref[...], 0.0)
            o = jnp.dot(hd, wm2_ref[...], preferred_element_type=jnp.float32)
            out_ref[...] = o + bm2_ref[...]


def _layer3_pool_tc(h2, batch3, w1, b1, gamma, beta, w2, b2,
                    wm1, bm1, wm2, bm2):
    return pl.pallas_call(
        _k12p_body,
        grid=(2, NB),
        in_specs=[
            pl.BlockSpec((NC, BLK, HALF),
                         lambda ph, i: (0, jnp.where(ph == 0, i, 0), 0)),
            pl.BlockSpec((1, 1, BLK),
                         lambda ph, i: (jnp.where(ph == 1, i, 0), 0, 0)),
            _const_spec((H, H)),
            _const_spec((1, H)),
            _const_spec((1, H)),
            _const_spec((1, H)),
            _const_spec((H, H)),
            _const_spec((1, H)),
            _const_spec((H, H)),
            _const_spec((1, H)),
            _const_spec((H, OUT)),
            _const_spec((1, OUT)),
        ],
        out_specs=[
            _const_spec((G, OUT)),
            _const_spec((G, H)),
        ],
        out_shape=[
            jax.ShapeDtypeStruct((G, OUT), jnp.float32),
            jax.ShapeDtypeStruct((G, H), jnp.float32),
        ],
        scratch_shapes=[
            pltpu.VMEM((N, H), jnp.float32),
            pltpu.VMEM((8, H), jnp.float32),
        ],
        compiler_params=pltpu.CompilerParams(
            dimension_semantics=("arbitrary", "arbitrary")),
    )(h2, batch3, w1, b1, gamma, beta, w2, b2, wm1, bm1, wm2, bm2)


def _agg_call(x2, src3, dst3):
    return _make_sc_agg()(x2, src3, dst3)


def kernel(x, edge_index, edge_attr, batch, conv_params, mlp_params):
    npad = EPAD - E
    pad_ids = jnp.arange(npad, dtype=jnp.int32)
    src_p = jnp.concatenate([edge_index[0], pad_ids % N])
    dst_p = jnp.concatenate([edge_index[1], N + pad_ids % NDUM])
    src3 = src_p.reshape(NS, NBK, CPB, CH)
    dst3 = dst_p.reshape(NS, NBK, CPB, CH)
    batch3 = batch.reshape(NB, 1, BLK)

    x2 = _split(x)
    for (w1, b1, gamma, beta, w2, b2) in conv_params[:-1]:
        h2 = _agg_call(x2, src3, dst3)
        x2 = _layer_tc(h2, w1, b1.reshape(1, H), gamma.reshape(1, H),
                       beta.reshape(1, H), w2, b2.reshape(1, H))

    (w1, b1, gamma, beta, w2, b2) = conv_params[-1]
    wm1, bm1, wm2, bm2 = mlp_params
    h2 = _agg_call(x2, src3, dst3)
    out, pooled = _layer3_pool_tc(
        h2, batch3, w1, b1.reshape(1, H), gamma.reshape(1, H),
        beta.reshape(1, H), w2, b2.reshape(1, H),
        wm1, bm1.reshape(1, H), wm2, bm2.reshape(1, OUT))
    return out, pooled
